# baseline (speedup 1.0000x reference)
import functools

import jax
import jax.numpy as jnp
from jax import lax
from jax.experimental import pallas as pl
from jax.experimental.pallas import tpu as pltpu
from jax.experimental.pallas import tpu_sc as plsc

N = 100000
E = 1600000
NP = 100352
H = 64
G = 64
C = 10

NBKT = 4
BKT = 25000
ACC_ROWS = 25024
DUMP = 25000
NW = 32
EW = E // NW
CH = 2048
NFULL = EW // CH
TAIL = EW - NFULL * CH
NBATCH_CAP = (EW + 127) // 128 + 9
NB3 = NW * NBKT * NBATCH_CAP
PCAP = 2304

RB = 2048
GRID = NP // RB


def _sc_bin_body(src_hbm, dst_hbm, sbdb_hbm, nbt_hbm, degp_hbm,
                 srcbuf, dstbuf, ps0, ps1, ps2, ps3, pd0, pd1, pd2, pd3,
                 didx, ones_v, nbuf, zbuf, degsp):
    pend_s = [ps0, ps1, ps2, ps3]
    pend_d = [pd0, pd1, pd2, pd3]
    c = lax.axis_index("c")
    s = lax.axis_index("s")
    w = 2 * s + c
    iota = lax.iota(jnp.int32, 16)
    z16f = jnp.zeros((16,), jnp.float32)

    def _zf(i, _):
        zbuf[pl.ds(i * 16, 16)] = z16f
        return 0
    lax.fori_loop(0, CH // 16, _zf, 0)
    for kk in range(3):
        pltpu.sync_copy(zbuf, degsp.at[pl.ds(s * 6272 + kk * CH, CH)])
    pltpu.sync_copy(zbuf.at[pl.ds(0, 128)],
                    degsp.at[pl.ds(s * 6272 + 3 * CH, 128)])
    for i in range(8):
        ones_v[pl.ds(i * 16, 16)] = jnp.ones((16,), jnp.float32)
    plsc.subcore_barrier()

    def process_chunk(estart, sz, carry):
        rem = list(carry[0:4])
        nb = list(carry[4:8])
        pltpu.sync_copy(src_hbm.at[pl.ds(estart, sz)], srcbuf.at[pl.ds(0, sz)])
        pltpu.sync_copy(dst_hbm.at[pl.ds(estart, sz)], dstbuf.at[pl.ds(0, sz)])

        nsb = sz // 128
        for j in range(nsb):
            for kk in range(8):
                didx[0, pl.ds(kk * 16, 16)] = dstbuf[pl.ds(j * 128 + kk * 16, 16)]
            pltpu.sync_copy(ones_v, degsp.at[didx.at[0]], add=True)
        tail = sz - nsb * 128
        if tail:
            dump16 = jnp.full((16,), N, jnp.int32)
            for kk in range(8):
                if kk * 16 < tail:
                    didx[0, pl.ds(kk * 16, 16)] = dstbuf[
                        pl.ds(nsb * 128 + kk * 16, 16)]
                else:
                    didx[0, pl.ds(kk * 16, 16)] = dump16
            pltpu.sync_copy(ones_v, degsp.at[didx.at[0]], add=True)

        def _vreg(k, fills):
            f0, f1, f2, f3 = fills
            sv = srcbuf[pl.ds(k * 16, 16)]
            dv = dstbuf[pl.ds(k * 16, 16)]
            bid = ((dv >= BKT).astype(jnp.int32)
                   + (dv >= 2 * BKT).astype(jnp.int32)
                   + (dv >= 3 * BKT).astype(jnp.int32))
            dloc = dv - bid * BKT
            fl = [f0, f1, f2, f3]
            for b in range(NBKT):
                m = bid == b
                plsc.store_compressed(pend_s[b].at[pl.ds(fl[b], 16)], sv,
                                      mask=m)
                plsc.store_compressed(pend_d[b].at[pl.ds(fl[b], 16)], dloc,
                                      mask=m)
                fl[b] = fl[b] + jnp.sum(m.astype(jnp.int32))
            return tuple(fl)
        fills = lax.fori_loop(0, sz // 16, _vreg,
                              (rem[0], rem[1], rem[2], rem[3]))
        fills = list(fills)

        for b in range(NBKT):
            nfl = fills[b] // 128
            rbase = (w * NBKT + b) * NBATCH_CAP

            def _flush(k, nbb):
                bi = rbase + nbb
                pltpu.sync_copy(pend_s[b].at[pl.ds(k * 128, 128)],
                                sbdb_hbm.at[bi, 0])
                pltpu.sync_copy(pend_d[b].at[pl.ds(k * 128, 128)],
                                sbdb_hbm.at[bi, 1])
                return nbb + 1
            nb[b] = lax.fori_loop(0, nfl, _flush, nb[b])
            base = nfl * 128
            for kk in range(8):
                v = pend_s[b][pl.ds(base + kk * 16, 16)]
                pend_s[b][pl.ds(kk * 16, 16)] = v
                v2 = pend_d[b][pl.ds(base + kk * 16, 16)]
                pend_d[b][pl.ds(kk * 16, 16)] = v2
            fills[b] = fills[b] - nfl * 128
        return tuple(fills) + tuple(nb)

    zs = jnp.zeros((), jnp.int32)
    carry = (zs, zs, zs, zs, zs, zs, zs, zs)

    def _chunk(ci, carry):
        return process_chunk(w * EW + ci * CH, CH, carry)
    carry = lax.fori_loop(0, NFULL, _chunk, carry)
    carry = process_chunk(w * EW + NFULL * CH, TAIL, carry)
    rem = list(carry[0:4])
    nb = list(carry[4:8])

    nbv = jnp.zeros((16,), jnp.int32)
    dump16 = jnp.full((16,), DUMP, jnp.int32)
    z16i = jnp.zeros((16,), jnp.int32)
    for b in range(NBKT):
        for kk in range(8):
            idx = iota + kk * 16
            mpad = idx >= rem[b]
            v = pend_s[b][pl.ds(kk * 16, 16)]
            pend_s[b][pl.ds(kk * 16, 16)] = jnp.where(mpad, 0, v)
            v2 = pend_d[b][pl.ds(kk * 16, 16)]
            pend_d[b][pl.ds(kk * 16, 16)] = jnp.where(mpad, DUMP, v2)
            pend_s[b][pl.ds(128 + kk * 16, 16)] = z16i
            pend_d[b][pl.ds(128 + kk * 16, 16)] = dump16
        rbase = (w * NBKT + b) * NBATCH_CAP
        bi = rbase + nb[b]
        pltpu.sync_copy(pend_s[b].at[pl.ds(0, 128)], sbdb_hbm.at[bi, 0])
        pltpu.sync_copy(pend_d[b].at[pl.ds(0, 128)], sbdb_hbm.at[bi, 1])
        for t in range(8):
            pltpu.sync_copy(pend_s[b].at[pl.ds(128, 128)],
                            sbdb_hbm.at[bi + 1 + t, 0])
            pltpu.sync_copy(pend_d[b].at[pl.ds(128, 128)],
                            sbdb_hbm.at[bi + 1 + t, 1])
        nbv = nbv + jnp.where(iota == b, nb[b] + 1, 0)

    nbuf[...] = nbv
    pltpu.sync_copy(nbuf, nbt_hbm.at[w])

    plsc.subcore_barrier()
    pltpu.sync_copy(degsp.at[pl.ds(s * 6272, 6272)],
                    degp_hbm.at[c, pl.ds(s * 6272, 6272)])


def _make_sc_bin():
    mesh = plsc.VectorSubcoreMesh(core_axis_name="c", subcore_axis_name="s",
                                  num_cores=2, num_subcores=16)
    return functools.partial(
        pl.kernel,
        mesh=mesh,
        compiler_params=pltpu.CompilerParams(needs_layout_passes=False, use_tc_tiling_on_sc=False),
        out_type=[
            jax.ShapeDtypeStruct((NB3, 2, 128), jnp.int32),
            jax.ShapeDtypeStruct((NW, 16), jnp.int32),
            jax.ShapeDtypeStruct((2, NP), jnp.float32),
        ],
        scratch_types=[
            pltpu.VMEM((CH,), jnp.int32),
            pltpu.VMEM((CH,), jnp.int32),
            pltpu.VMEM((PCAP,), jnp.int32),
            pltpu.VMEM((PCAP,), jnp.int32),
            pltpu.VMEM((PCAP,), jnp.int32),
            pltpu.VMEM((PCAP,), jnp.int32),
            pltpu.VMEM((PCAP,), jnp.int32),
            pltpu.VMEM((PCAP,), jnp.int32),
            pltpu.VMEM((PCAP,), jnp.int32),
            pltpu.VMEM((PCAP,), jnp.int32),
            pltpu.VMEM((1, 128), jnp.int32),
            pltpu.VMEM((128,), jnp.float32),
            pltpu.VMEM((16,), jnp.int32),
            pltpu.VMEM((CH,), jnp.float32),
            pltpu.VMEM_SHARED((NP,), jnp.float32),
        ],
    )(_sc_bin_body)


def _sc_prop_body(g_hbm, sbdb_hbm, nbt_hbm, out_hbm,
                  ib0, ib1, ib2, rows0, rows1, rows2, nbuf,
                  sem0, sem1, sem2, acc):
    c = lax.axis_index("c")
    s = lax.axis_index("s")
    iota = lax.iota(jnp.int32, 16)
    z16 = jnp.zeros((16,), jnp.float32)

    def _z(i, _):
        r = i // 4
        k = i % 4
        rows0[r, pl.ds(k * 16, 16)] = z16
        return 0
    lax.fori_loop(0, 128 * 4, _z, 0)

    for j in range(2):
        b = 2 * c + j
        for k in range(12):
            pltpu.sync_copy(rows0, acc.at[pl.ds(s * 1564 + k * 128, 128)])
        pltpu.sync_copy(rows0.at[pl.ds(0, 28)],
                        acc.at[pl.ds(s * 1564 + 12 * 128, 28)])
        plsc.subcore_barrier()

        for jj in range(2):
            wk = 2 * s + jj
            pltpu.sync_copy(nbt_hbm.at[wk], nbuf)
            nv = nbuf[pl.ds(0, 16)]
            nbatch = jnp.sum(jnp.where(iota == b, nv, 0))
            rbase = (wk * NBKT + b) * NBATCH_CAP

            def _trip(p, _):
                k0 = 3 * p
                k1 = jnp.minimum(k0 + 1, nbatch)
                k2 = jnp.minimum(k0 + 2, nbatch)
                pltpu.sync_copy(sbdb_hbm.at[k0 + rbase], ib0)
                cp0 = pltpu.async_copy(g_hbm.at[ib0.at[0]], rows0, sem0)
                pltpu.sync_copy(sbdb_hbm.at[k1 + rbase], ib1)
                cp1 = pltpu.async_copy(g_hbm.at[ib1.at[0]], rows1, sem1)
                pltpu.sync_copy(sbdb_hbm.at[k2 + rbase], ib2)
                cp2 = pltpu.async_copy(g_hbm.at[ib2.at[0]], rows2, sem2)
                cp0.wait()
                pltpu.sync_copy(rows0, acc.at[ib0.at[1]], add=True)
                cp1.wait()
                pltpu.sync_copy(rows1, acc.at[ib1.at[1]], add=True)
                cp2.wait()
                pltpu.sync_copy(rows2, acc.at[ib2.at[1]], add=True)
                return 0
            lax.fori_loop(0, (nbatch + 2) // 3, _trip, 0)

        if j == 0:
            def _z2(i, _):
                r = i // 4
                k = i % 4
                rows0[r, pl.ds(k * 16, 16)] = z16
                return 0
            lax.fori_loop(0, 128 * 4, _z2, 0)
        plsc.subcore_barrier()
        pltpu.sync_copy(acc.at[pl.ds(s * 1560, 1560)],
                        out_hbm.at[pl.ds(b * BKT + s * 1560, 1560)])
        @pl.when(s == 15)
        def _():
            pltpu.sync_copy(acc.at[pl.ds(24960, 40)],
                            out_hbm.at[pl.ds(b * BKT + 24960, 40)])
        plsc.subcore_barrier()


def _make_sc_prop():
    mesh = plsc.VectorSubcoreMesh(core_axis_name="c", subcore_axis_name="s",
                                  num_cores=2, num_subcores=16)
    return functools.partial(
        pl.kernel,
        mesh=mesh,
        compiler_params=pltpu.CompilerParams(needs_layout_passes=False, use_tc_tiling_on_sc=False),
        out_type=jax.ShapeDtypeStruct((NP, H), jnp.float32),
        scratch_types=[
            pltpu.VMEM((2, 128), jnp.int32),
            pltpu.VMEM((2, 128), jnp.int32),
            pltpu.VMEM((2, 128), jnp.int32),
            pltpu.VMEM((128, H), jnp.float32),
            pltpu.VMEM((128, H), jnp.float32),
            pltpu.VMEM((128, H), jnp.float32),
            pltpu.VMEM((16,), jnp.int32),
            pltpu.SemaphoreType.DMA,
            pltpu.SemaphoreType.DMA,
            pltpu.SemaphoreType.DMA,
            pltpu.VMEM_SHARED((ACC_ROWS, H), jnp.float32),
        ],
    )(_sc_prop_body)


def _prep_body(dp0, dp1, x, w0, dinv_o, g0_o):
    deg = dp0[...] + dp1[...] + 1.0
    di = lax.rsqrt(deg)
    z = jnp.dot(x[...], w0[...], preferred_element_type=jnp.float32)
    dinv_o[...] = di
    g0_o[...] = z * di


def _prep(dp0, dp1, xp, w0):
    return pl.pallas_call(
        _prep_body,
        grid=(GRID,),
        in_specs=[
            pl.BlockSpec((RB, 1), lambda i: (i, 0)),
            pl.BlockSpec((RB, 1), lambda i: (i, 0)),
            pl.BlockSpec((RB, 3), lambda i: (i, 0)),
            pl.BlockSpec((3, H), lambda i: (0, 0)),
        ],
        out_specs=[
            pl.BlockSpec((RB, 1), lambda i: (i, 0)),
            pl.BlockSpec((RB, H), lambda i: (i, 0)),
        ],
        out_shape=[
            jax.ShapeDtypeStruct((NP, 1), jnp.float32),
            jax.ShapeDtypeStruct((NP, H), jnp.float32),
        ],
    )(dp0, dp1, xp, w0)


def _mid_body(s_in, g_in, dinv, wn, bc, g_next):
    i = pl.program_id(0)
    di = dinv[...]
    h = (s_in[...] + g_in[...]) * di + bc[...]
    h = jnp.maximum(h, 0.0)
    rid = i * RB + lax.broadcasted_iota(jnp.int32, (RB, 1), 0)
    h = jnp.where(rid < N, h, 0.0)
    g_next[...] = jnp.dot(h, wn[...], preferred_element_type=jnp.float32) * di


def _mid(s_arr, g_arr, dinv, wn, bc):
    return pl.pallas_call(
        _mid_body,
        grid=(GRID,),
        in_specs=[
            pl.BlockSpec((RB, H), lambda i: (i, 0)),
            pl.BlockSpec((RB, H), lambda i: (i, 0)),
            pl.BlockSpec((RB, 1), lambda i: (i, 0)),
            pl.BlockSpec((H, H), lambda i: (0, 0)),
            pl.BlockSpec((1, H), lambda i: (0, 0)),
        ],
        out_specs=pl.BlockSpec((RB, H), lambda i: (i, 0)),
        out_shape=jax.ShapeDtypeStruct((NP, H), jnp.float32),
    )(s_arr, g_arr, dinv, wn, bc)


def _pool_body(s_in, g_in, dinv, bc, bat, sums, maxs, cnts):
    i = pl.program_id(0)

    @pl.when(i == 0)
    def _():
        sums[...] = jnp.zeros((G, H), jnp.float32)
        maxs[...] = jnp.full((G, H), -3.0e38, jnp.float32)
        cnts[...] = jnp.zeros((G, 1), jnp.float32)

    h3 = (s_in[...] + g_in[...]) * dinv[...] + bc[...]
    rid = i * RB + lax.broadcasted_iota(jnp.int32, (RB, 1), 0)
    h3 = jnp.where(rid < N, h3, 0.0)
    b = bat[...]
    oh = (b == lax.broadcasted_iota(jnp.int32, (RB, G), 1)).astype(jnp.float32)
    sums[...] += lax.dot_general(oh, h3, (((0,), (0,)), ((), ())),
                                 preferred_element_type=jnp.float32)
    cnts[...] += lax.dot_general(oh, jnp.ones((RB, 1), jnp.float32),
                                 (((0,), (0,)), ((), ())),
                                 preferred_element_type=jnp.float32)
    glo = jnp.min(b)
    ghi = jnp.minimum(jnp.max(b), G - 1)

    def _seg(g, _):
        mcol = b == g
        vals = jnp.where(mcol, h3, -3.0e38)
        vmax = jnp.max(vals, axis=0, keepdims=True)
        cur = maxs[pl.ds(g, 1), :]
        maxs[pl.ds(g, 1), :] = jnp.maximum(cur, vmax)
        return 0
    lax.fori_loop(glo, ghi + 1, _seg, 0)


def _pool(s_arr, g_arr, dinv, bc, batp):
    return pl.pallas_call(
        _pool_body,
        grid=(GRID,),
        in_specs=[
            pl.BlockSpec((RB, H), lambda i: (i, 0)),
            pl.BlockSpec((RB, H), lambda i: (i, 0)),
            pl.BlockSpec((RB, 1), lambda i: (i, 0)),
            pl.BlockSpec((1, H), lambda i: (0, 0)),
            pl.BlockSpec((RB, 1), lambda i: (i, 0)),
        ],
        out_specs=[
            pl.BlockSpec((G, H), lambda i: (0, 0)),
            pl.BlockSpec((G, H), lambda i: (0, 0)),
            pl.BlockSpec((G, 1), lambda i: (0, 0)),
        ],
        out_shape=[
            jax.ShapeDtypeStruct((G, H), jnp.float32),
            jax.ShapeDtypeStruct((G, H), jnp.float32),
            jax.ShapeDtypeStruct((G, 1), jnp.float32),
        ],
    )(s_arr, g_arr, dinv, bc, batp)


def _head_body(sums, maxs, cnts, f1a, f1b, fb1, f2, fb2, out):
    cn = cnts[...]
    mean = sums[...] / jnp.maximum(cn, 1.0)
    mx = jnp.where(cn > 0.0, maxs[...], 0.0)
    a = jnp.dot(mean, f1a[...], preferred_element_type=jnp.float32)
    a += jnp.dot(mx, f1b[...], preferred_element_type=jnp.float32)
    a = jnp.maximum(a + fb1[...], 0.0)
    o = jnp.dot(a, f2[...], preferred_element_type=jnp.float32) + fb2[...]
    m = jnp.max(o, axis=1, keepdims=True)
    ex = jnp.exp(o - m)
    lse = jnp.log(jnp.sum(ex, axis=1, keepdims=True)) + m
    out[...] = o - lse


def _head(sums, maxs, cnts, f1a, f1b, fb1, f2, fb2):
    return pl.pallas_call(
        _head_body,
        out_shape=jax.ShapeDtypeStruct((G, C), jnp.float32),
    )(sums, maxs, cnts, f1a, f1b, fb1, f2, fb2)


def kernel(x, edge_index, batch, W0, b0, W1, b1, W2, b2,
           bn_gamma, bn_beta, bn_mean, bn_var, fc1_W, fc1_b, fc2_W, fc2_b):
    src = edge_index[0]
    dst = edge_index[1]

    gp = bn_gamma * lax.rsqrt(bn_var + 1e-5)
    Wt0 = W0 * gp[0][None, :]
    Wt1 = W1 * gp[1][None, :]
    Wt2 = W2 * gp[2][None, :]
    bt0 = ((b0 - bn_mean[0]) * gp[0] + bn_beta[0])[None, :]
    bt1 = ((b1 - bn_mean[1]) * gp[1] + bn_beta[1])[None, :]
    bt2 = ((b2 - bn_mean[2]) * gp[2] + bn_beta[2])[None, :]

    xp = jnp.pad(x, ((0, NP - N), (0, 0)))
    batp = jnp.pad(batch, (0, NP - N), constant_values=G).reshape(NP, 1)

    sc_bin = _make_sc_bin()
    sc_prop = _make_sc_prop()

    sbdb, nbt, degp = sc_bin(src, dst)
    dp0 = degp[0].reshape(NP, 1)
    dp1 = degp[1].reshape(NP, 1)
    dinv, g = _prep(dp0, dp1, xp, Wt0)

    s0 = sc_prop(g, sbdb, nbt)
    g = _mid(s0, g, dinv, Wt1, bt0)
    s1 = sc_prop(g, sbdb, nbt)
    g = _mid(s1, g, dinv, Wt2, bt1)
    s2 = sc_prop(g, sbdb, nbt)

    sums, maxs, cnts = _pool(s2, g, dinv, bt2, batp)
    out = _head(sums, maxs, cnts, fc1_W[:H], fc1_W[H:], fc1_b[None, :],
                fc2_W, fc2_b[None, :])
    return out


# speedup vs baseline: 1.1860x; 1.1860x over previous
import functools

import jax
import jax.numpy as jnp
from jax import lax
from jax.experimental import pallas as pl
from jax.experimental.pallas import tpu as pltpu
from jax.experimental.pallas import tpu_sc as plsc

N = 100000
E = 1600000
NP = 100352
H = 64
G = 64
C = 10

NBKT = 4
BKT = 25000
ACC_ROWS = 25024
DUMP = 25000
NW = 32
EW = E // NW
CH = 2048
NFULL = EW // CH
TAIL = EW - NFULL * CH
NBATCH_CAP = (EW + 127) // 128 + 9
NB3 = NW * NBKT * NBATCH_CAP
PCAP = 2304

RB = 2048
GRID = NP // RB


def _sc_bin_body(src_hbm, dst_hbm, sbdb_hbm, nbt_hbm, degp_hbm,
                 srcbuf, dstbuf, ps0, ps1, ps2, ps3, pd0, pd1, pd2, pd3,
                 didx, ones_v, nbuf, zbuf, degsp):
    pend_s = [ps0, ps1, ps2, ps3]
    pend_d = [pd0, pd1, pd2, pd3]
    c = lax.axis_index("c")
    s = lax.axis_index("s")
    w = 2 * s + c
    iota = lax.iota(jnp.int32, 16)
    z16f = jnp.zeros((16,), jnp.float32)

    def _zf(i, _):
        zbuf[pl.ds(i * 16, 16)] = z16f
        return 0
    lax.fori_loop(0, CH // 16, _zf, 0)
    for kk in range(3):
        pltpu.sync_copy(zbuf, degsp.at[pl.ds(s * 6272 + kk * CH, CH)])
    pltpu.sync_copy(zbuf.at[pl.ds(0, 128)],
                    degsp.at[pl.ds(s * 6272 + 3 * CH, 128)])
    for i in range(8):
        ones_v[pl.ds(i * 16, 16)] = jnp.ones((16,), jnp.float32)
    plsc.subcore_barrier()

    def process_chunk(estart, sz, carry):
        rem = list(carry[0:4])
        nb = list(carry[4:8])
        pltpu.sync_copy(src_hbm.at[pl.ds(estart, sz)], srcbuf.at[pl.ds(0, sz)])
        pltpu.sync_copy(dst_hbm.at[pl.ds(estart, sz)], dstbuf.at[pl.ds(0, sz)])

        nsb = sz // 128
        for j in range(nsb):
            for kk in range(8):
                didx[0, pl.ds(kk * 16, 16)] = dstbuf[pl.ds(j * 128 + kk * 16, 16)]
            pltpu.sync_copy(ones_v, degsp.at[didx.at[0]], add=True)
        tail = sz - nsb * 128
        if tail:
            dump16 = jnp.full((16,), N, jnp.int32)
            for kk in range(8):
                if kk * 16 < tail:
                    didx[0, pl.ds(kk * 16, 16)] = dstbuf[
                        pl.ds(nsb * 128 + kk * 16, 16)]
                else:
                    didx[0, pl.ds(kk * 16, 16)] = dump16
            pltpu.sync_copy(ones_v, degsp.at[didx.at[0]], add=True)

        def _vreg(k, fills):
            f0, f1, f2, f3 = fills
            sv = srcbuf[pl.ds(k * 16, 16)]
            dv = dstbuf[pl.ds(k * 16, 16)]
            bid = ((dv >= BKT).astype(jnp.int32)
                   + (dv >= 2 * BKT).astype(jnp.int32)
                   + (dv >= 3 * BKT).astype(jnp.int32))
            dloc = dv - bid * BKT
            fl = [f0, f1, f2, f3]
            for b in range(NBKT):
                m = bid == b
                plsc.store_compressed(pend_s[b].at[pl.ds(fl[b], 16)], sv,
                                      mask=m)
                plsc.store_compressed(pend_d[b].at[pl.ds(fl[b], 16)], dloc,
                                      mask=m)
                fl[b] = fl[b] + jnp.sum(m.astype(jnp.int32))
            return tuple(fl)
        fills = lax.fori_loop(0, sz // 16, _vreg,
                              (rem[0], rem[1], rem[2], rem[3]))
        fills = list(fills)

        for b in range(NBKT):
            nfl = fills[b] // 128
            rbase = (w * NBKT + b) * NBATCH_CAP

            def _flush(k, nbb):
                bi = rbase + nbb
                pltpu.sync_copy(pend_s[b].at[pl.ds(k * 128, 128)],
                                sbdb_hbm.at[bi, 0])
                pltpu.sync_copy(pend_d[b].at[pl.ds(k * 128, 128)],
                                sbdb_hbm.at[bi, 1])
                return nbb + 1
            nb[b] = lax.fori_loop(0, nfl, _flush, nb[b])
            base = nfl * 128
            for kk in range(8):
                v = pend_s[b][pl.ds(base + kk * 16, 16)]
                pend_s[b][pl.ds(kk * 16, 16)] = v
                v2 = pend_d[b][pl.ds(base + kk * 16, 16)]
                pend_d[b][pl.ds(kk * 16, 16)] = v2
            fills[b] = fills[b] - nfl * 128
        return tuple(fills) + tuple(nb)

    zs = jnp.zeros((), jnp.int32)
    carry = (zs, zs, zs, zs, zs, zs, zs, zs)

    def _chunk(ci, carry):
        return process_chunk(w * EW + ci * CH, CH, carry)
    carry = lax.fori_loop(0, NFULL, _chunk, carry)
    carry = process_chunk(w * EW + NFULL * CH, TAIL, carry)
    rem = list(carry[0:4])
    nb = list(carry[4:8])

    nbv = jnp.zeros((16,), jnp.int32)
    dump16 = jnp.full((16,), DUMP, jnp.int32)
    z16i = jnp.zeros((16,), jnp.int32)
    for b in range(NBKT):
        for kk in range(8):
            idx = iota + kk * 16
            mpad = idx >= rem[b]
            v = pend_s[b][pl.ds(kk * 16, 16)]
            pend_s[b][pl.ds(kk * 16, 16)] = jnp.where(mpad, 0, v)
            v2 = pend_d[b][pl.ds(kk * 16, 16)]
            pend_d[b][pl.ds(kk * 16, 16)] = jnp.where(mpad, DUMP, v2)
            pend_s[b][pl.ds(128 + kk * 16, 16)] = z16i
            pend_d[b][pl.ds(128 + kk * 16, 16)] = dump16
        rbase = (w * NBKT + b) * NBATCH_CAP
        bi = rbase + nb[b]
        pltpu.sync_copy(pend_s[b].at[pl.ds(0, 128)], sbdb_hbm.at[bi, 0])
        pltpu.sync_copy(pend_d[b].at[pl.ds(0, 128)], sbdb_hbm.at[bi, 1])
        for t in range(8):
            pltpu.sync_copy(pend_s[b].at[pl.ds(128, 128)],
                            sbdb_hbm.at[bi + 1 + t, 0])
            pltpu.sync_copy(pend_d[b].at[pl.ds(128, 128)],
                            sbdb_hbm.at[bi + 1 + t, 1])
        nbv = nbv + jnp.where(iota == b, nb[b] + 1, 0)

    nbuf[...] = nbv
    pltpu.sync_copy(nbuf, nbt_hbm.at[w])

    plsc.subcore_barrier()
    pltpu.sync_copy(degsp.at[pl.ds(s * 6272, 6272)],
                    degp_hbm.at[c, pl.ds(s * 6272, 6272)])


def _make_sc_bin():
    mesh = plsc.VectorSubcoreMesh(core_axis_name="c", subcore_axis_name="s",
                                  num_cores=2, num_subcores=16)
    return functools.partial(
        pl.kernel,
        mesh=mesh,
        compiler_params=pltpu.CompilerParams(needs_layout_passes=False, use_tc_tiling_on_sc=False),
        out_type=[
            jax.ShapeDtypeStruct((NB3, 2, 128), jnp.int32),
            jax.ShapeDtypeStruct((NW, 16), jnp.int32),
            jax.ShapeDtypeStruct((2, NP), jnp.float32),
        ],
        scratch_types=[
            pltpu.VMEM((CH,), jnp.int32),
            pltpu.VMEM((CH,), jnp.int32),
            pltpu.VMEM((PCAP,), jnp.int32),
            pltpu.VMEM((PCAP,), jnp.int32),
            pltpu.VMEM((PCAP,), jnp.int32),
            pltpu.VMEM((PCAP,), jnp.int32),
            pltpu.VMEM((PCAP,), jnp.int32),
            pltpu.VMEM((PCAP,), jnp.int32),
            pltpu.VMEM((PCAP,), jnp.int32),
            pltpu.VMEM((PCAP,), jnp.int32),
            pltpu.VMEM((1, 128), jnp.int32),
            pltpu.VMEM((128,), jnp.float32),
            pltpu.VMEM((16,), jnp.int32),
            pltpu.VMEM((CH,), jnp.float32),
            pltpu.VMEM_SHARED((NP,), jnp.float32),
        ],
    )(_sc_bin_body)


def _sc_prop_body(zz_hbm, g_hbm, sbdb_hbm, nbt_hbm, out_hbm,
                  ib0, ib1, rows0, rows1, nbuf, sem0, sem1, acc):
    c = lax.axis_index("c")
    s = lax.axis_index("s")
    iota = lax.iota(jnp.int32, 16)

    for j in range(2):
        b = 2 * c + j
        pltpu.sync_copy(zz_hbm, rows0)
        for k in range(12):
            pltpu.sync_copy(rows0, acc.at[pl.ds(s * 1564 + k * 128, 128)])
        pltpu.sync_copy(rows0.at[pl.ds(0, 28)],
                        acc.at[pl.ds(s * 1564 + 12 * 128, 28)])
        plsc.subcore_barrier()

        for jj in range(2):
            wk = 2 * s + jj
            pltpu.sync_copy(nbt_hbm.at[wk], nbuf)
            nv = nbuf[pl.ds(0, 16)]
            nbatch = jnp.sum(jnp.where(iota == b, nv, 0))
            rbase = (wk * NBKT + b) * NBATCH_CAP

            def _pair(p, _):
                k0 = 2 * p
                k1 = jnp.minimum(k0 + 1, nbatch)
                pltpu.sync_copy(sbdb_hbm.at[k0 + rbase], ib0)
                cp0 = pltpu.async_copy(g_hbm.at[ib0.at[0]], rows0, sem0)
                pltpu.sync_copy(sbdb_hbm.at[k1 + rbase], ib1)
                cp1 = pltpu.async_copy(g_hbm.at[ib1.at[0]], rows1, sem1)
                cp0.wait()
                pltpu.sync_copy(rows0, acc.at[ib0.at[1]], add=True)
                cp1.wait()
                pltpu.sync_copy(rows1, acc.at[ib1.at[1]], add=True)
                return 0
            lax.fori_loop(0, (nbatch + 1) // 2, _pair, 0)

        plsc.subcore_barrier()
        pltpu.sync_copy(acc.at[pl.ds(s * 1560, 1560)],
                        out_hbm.at[pl.ds(b * BKT + s * 1560, 1560)])
        @pl.when(s == 15)
        def _():
            pltpu.sync_copy(acc.at[pl.ds(24960, 40)],
                            out_hbm.at[pl.ds(b * BKT + 24960, 40)])
        plsc.subcore_barrier()


def _make_sc_prop(dd):
    mesh = plsc.VectorSubcoreMesh(core_axis_name="c", subcore_axis_name="s",
                                  num_cores=2, num_subcores=16)
    return functools.partial(
        pl.kernel,
        mesh=mesh,
        compiler_params=pltpu.CompilerParams(needs_layout_passes=False, use_tc_tiling_on_sc=False),
        out_type=jax.ShapeDtypeStruct((NP, dd), jnp.float32),
        scratch_types=[
            pltpu.VMEM((2, 128), jnp.int32),
            pltpu.VMEM((2, 128), jnp.int32),
            pltpu.VMEM((128, dd), jnp.float32),
            pltpu.VMEM((128, dd), jnp.float32),
            pltpu.VMEM((16,), jnp.int32),
            pltpu.SemaphoreType.DMA,
            pltpu.SemaphoreType.DMA,
            pltpu.VMEM_SHARED((ACC_ROWS, dd), jnp.float32),
        ],
    )(_sc_prop_body)


def _prep_body(dp0, dp1, x, dinv_o, g0_o):
    deg = dp0[...] + dp1[...] + 1.0
    di = lax.rsqrt(deg)
    dinv_o[...] = di
    g0_o[...] = x[...] * di


def _prep(dp0, dp1, xp4):
    return pl.pallas_call(
        _prep_body,
        grid=(GRID,),
        in_specs=[
            pl.BlockSpec((RB, 1), lambda i: (i, 0)),
            pl.BlockSpec((RB, 1), lambda i: (i, 0)),
            pl.BlockSpec((RB, 16), lambda i: (i, 0)),
        ],
        out_specs=[
            pl.BlockSpec((RB, 1), lambda i: (i, 0)),
            pl.BlockSpec((RB, 16), lambda i: (i, 0)),
        ],
        out_shape=[
            jax.ShapeDtypeStruct((NP, 1), jnp.float32),
            jax.ShapeDtypeStruct((NP, 16), jnp.float32),
        ],
    )(dp0, dp1, xp4)


def _mid0_body(s_in, g_in, dinv, w0, wn, bc, g_next):
    i = pl.program_id(0)
    di = dinv[...]
    h = jnp.dot((s_in[...] + g_in[...]) * di, w0[...],
                preferred_element_type=jnp.float32) + bc[...]
    h = jnp.maximum(h, 0.0)
    rid = i * RB + lax.broadcasted_iota(jnp.int32, (RB, 1), 0)
    h = jnp.where(rid < N, h, 0.0)
    g_next[...] = jnp.dot(h, wn[...], preferred_element_type=jnp.float32) * di


def _mid0(s_arr, g_arr, dinv, w0p, wn, bc):
    return pl.pallas_call(
        _mid0_body,
        grid=(GRID,),
        in_specs=[
            pl.BlockSpec((RB, 16), lambda i: (i, 0)),
            pl.BlockSpec((RB, 16), lambda i: (i, 0)),
            pl.BlockSpec((RB, 1), lambda i: (i, 0)),
            pl.BlockSpec((16, H), lambda i: (0, 0)),
            pl.BlockSpec((H, H), lambda i: (0, 0)),
            pl.BlockSpec((1, H), lambda i: (0, 0)),
        ],
        out_specs=pl.BlockSpec((RB, H), lambda i: (i, 0)),
        out_shape=jax.ShapeDtypeStruct((NP, H), jnp.float32),
    )(s_arr, g_arr, dinv, w0p, wn, bc)


def _mid_body(s_in, g_in, dinv, wn, bc, g_next):
    i = pl.program_id(0)
    di = dinv[...]
    h = (s_in[...] + g_in[...]) * di + bc[...]
    h = jnp.maximum(h, 0.0)
    rid = i * RB + lax.broadcasted_iota(jnp.int32, (RB, 1), 0)
    h = jnp.where(rid < N, h, 0.0)
    g_next[...] = jnp.dot(h, wn[...], preferred_element_type=jnp.float32) * di


def _mid(s_arr, g_arr, dinv, wn, bc):
    return pl.pallas_call(
        _mid_body,
        grid=(GRID,),
        in_specs=[
            pl.BlockSpec((RB, H), lambda i: (i, 0)),
            pl.BlockSpec((RB, H), lambda i: (i, 0)),
            pl.BlockSpec((RB, 1), lambda i: (i, 0)),
            pl.BlockSpec((H, H), lambda i: (0, 0)),
            pl.BlockSpec((1, H), lambda i: (0, 0)),
        ],
        out_specs=pl.BlockSpec((RB, H), lambda i: (i, 0)),
        out_shape=jax.ShapeDtypeStruct((NP, H), jnp.float32),
    )(s_arr, g_arr, dinv, wn, bc)


def _pool_body(s_in, g_in, dinv, bc, bat, sums, maxs, cnts):
    i = pl.program_id(0)

    @pl.when(i == 0)
    def _():
        sums[...] = jnp.zeros((G, H), jnp.float32)
        maxs[...] = jnp.full((G, H), -3.0e38, jnp.float32)
        cnts[...] = jnp.zeros((G, 1), jnp.float32)

    h3 = (s_in[...] + g_in[...]) * dinv[...] + bc[...]
    rid = i * RB + lax.broadcasted_iota(jnp.int32, (RB, 1), 0)
    h3 = jnp.where(rid < N, h3, 0.0)
    b = bat[...]
    oh = (b == lax.broadcasted_iota(jnp.int32, (RB, G), 1)).astype(jnp.float32)
    sums[...] += lax.dot_general(oh, h3, (((0,), (0,)), ((), ())),
                                 preferred_element_type=jnp.float32)
    cnts[...] += lax.dot_general(oh, jnp.ones((RB, 1), jnp.float32),
                                 (((0,), (0,)), ((), ())),
                                 preferred_element_type=jnp.float32)
    glo = jnp.min(b)
    ghi = jnp.minimum(jnp.max(b), G - 1)

    def _seg(g, _):
        mcol = b == g
        vals = jnp.where(mcol, h3, -3.0e38)
        vmax = jnp.max(vals, axis=0, keepdims=True)
        cur = maxs[pl.ds(g, 1), :]
        maxs[pl.ds(g, 1), :] = jnp.maximum(cur, vmax)
        return 0
    lax.fori_loop(glo, ghi + 1, _seg, 0)


def _pool(s_arr, g_arr, dinv, bc, batp):
    return pl.pallas_call(
        _pool_body,
        grid=(GRID,),
        in_specs=[
            pl.BlockSpec((RB, H), lambda i: (i, 0)),
            pl.BlockSpec((RB, H), lambda i: (i, 0)),
            pl.BlockSpec((RB, 1), lambda i: (i, 0)),
            pl.BlockSpec((1, H), lambda i: (0, 0)),
            pl.BlockSpec((RB, 1), lambda i: (i, 0)),
        ],
        out_specs=[
            pl.BlockSpec((G, H), lambda i: (0, 0)),
            pl.BlockSpec((G, H), lambda i: (0, 0)),
            pl.BlockSpec((G, 1), lambda i: (0, 0)),
        ],
        out_shape=[
            jax.ShapeDtypeStruct((G, H), jnp.float32),
            jax.ShapeDtypeStruct((G, H), jnp.float32),
            jax.ShapeDtypeStruct((G, 1), jnp.float32),
        ],
    )(s_arr, g_arr, dinv, bc, batp)


def _head_body(sums, maxs, cnts, f1a, f1b, fb1, f2, fb2, out):
    cn = cnts[...]
    mean = sums[...] / jnp.maximum(cn, 1.0)
    mx = jnp.where(cn > 0.0, maxs[...], 0.0)
    a = jnp.dot(mean, f1a[...], preferred_element_type=jnp.float32)
    a += jnp.dot(mx, f1b[...], preferred_element_type=jnp.float32)
    a = jnp.maximum(a + fb1[...], 0.0)
    o = jnp.dot(a, f2[...], preferred_element_type=jnp.float32) + fb2[...]
    m = jnp.max(o, axis=1, keepdims=True)
    ex = jnp.exp(o - m)
    lse = jnp.log(jnp.sum(ex, axis=1, keepdims=True)) + m
    out[...] = o - lse


def _head(sums, maxs, cnts, f1a, f1b, fb1, f2, fb2):
    return pl.pallas_call(
        _head_body,
        out_shape=jax.ShapeDtypeStruct((G, C), jnp.float32),
    )(sums, maxs, cnts, f1a, f1b, fb1, f2, fb2)


def kernel(x, edge_index, batch, W0, b0, W1, b1, W2, b2,
           bn_gamma, bn_beta, bn_mean, bn_var, fc1_W, fc1_b, fc2_W, fc2_b):
    src = edge_index[0]
    dst = edge_index[1]

    gp = bn_gamma * lax.rsqrt(bn_var + 1e-5)
    Wt0 = W0 * gp[0][None, :]
    Wt1 = W1 * gp[1][None, :]
    Wt2 = W2 * gp[2][None, :]
    bt0 = ((b0 - bn_mean[0]) * gp[0] + bn_beta[0])[None, :]
    bt1 = ((b1 - bn_mean[1]) * gp[1] + bn_beta[1])[None, :]
    bt2 = ((b2 - bn_mean[2]) * gp[2] + bn_beta[2])[None, :]

    xp4 = jnp.pad(x, ((0, NP - N), (0, 13)))
    batp = jnp.pad(batch, (0, NP - N), constant_values=G).reshape(NP, 1)
    Wt0p = jnp.pad(Wt0, ((0, 13), (0, 0)))
    zz4 = jnp.zeros((128, 16), jnp.float32)
    zz64 = jnp.zeros((128, H), jnp.float32)

    sc_bin = _make_sc_bin()
    sc_prop4 = _make_sc_prop(16)
    sc_prop = _make_sc_prop(H)

    sbdb, nbt, degp = sc_bin(src, dst)
    dp0 = degp[0].reshape(NP, 1)
    dp1 = degp[1].reshape(NP, 1)
    dinv, g0x = _prep(dp0, dp1, xp4)

    s0 = sc_prop4(zz4, g0x, sbdb, nbt)
    g = _mid0(s0, g0x, dinv, Wt0p, Wt1, bt0)
    s1 = sc_prop(zz64, g, sbdb, nbt)
    g = _mid(s1, g, dinv, Wt2, bt1)
    s2 = sc_prop(zz64, g, sbdb, nbt)

    sums, maxs, cnts = _pool(s2, g, dinv, bt2, batp)
    out = _head(sums, maxs, cnts, fc1_W[:H], fc1_W[H:], fc1_b[None, :],
                fc2_W, fc2_b[None, :])
    return out


# speedup vs baseline: 1.1921x; 1.0052x over previous
import functools

import jax
import jax.numpy as jnp
from jax import lax
from jax.experimental import pallas as pl
from jax.experimental.pallas import tpu as pltpu
from jax.experimental.pallas import tpu_sc as plsc

N = 100000
E = 1600000
NP = 100352
H = 64
G = 64
C = 10

NBKT = 4
BKT = 25000
ACC_ROWS = 25024
DUMP = 25000
NW = 32
EW = E // NW
CH = 2048
NFULL = EW // CH
TAIL = EW - NFULL * CH
NBATCH_CAP = (EW + 127) // 128 + 9
NB3 = NW * NBKT * NBATCH_CAP
PCAP = 2304

RB = 2048
GRID = NP // RB


def _sc_bin_body(src_hbm, dst_hbm, sbdb_hbm, nbt_hbm, degp_hbm,
                 srcbuf, dstbuf, ps0, ps1, ps2, ps3, pd0, pd1, pd2, pd3,
                 didx, ones_v, nbuf, zbuf, dsem0, dsem1, degsp):
    pend_s = [ps0, ps1, ps2, ps3]
    pend_d = [pd0, pd1, pd2, pd3]
    c = lax.axis_index("c")
    s = lax.axis_index("s")
    w = 2 * s + c
    iota = lax.iota(jnp.int32, 16)
    z16f = jnp.zeros((16,), jnp.float32)

    def _zf(i, _):
        zbuf[pl.ds(i * 16, 16)] = z16f
        return 0
    lax.fori_loop(0, CH // 16, _zf, 0)
    for kk in range(3):
        pltpu.sync_copy(zbuf, degsp.at[pl.ds(s * 6272 + kk * CH, CH)])
    pltpu.sync_copy(zbuf.at[pl.ds(0, 128)],
                    degsp.at[pl.ds(s * 6272 + 3 * CH, 128)])
    for i in range(8):
        ones_v[pl.ds(i * 16, 16)] = jnp.ones((16,), jnp.float32)
    plsc.subcore_barrier()

    def process_chunk(estart, sz, carry):
        rem = list(carry[0:4])
        nb = list(carry[4:8])
        pltpu.sync_copy(src_hbm.at[pl.ds(estart, sz)], srcbuf.at[pl.ds(0, sz)])
        pltpu.sync_copy(dst_hbm.at[pl.ds(estart, sz)], dstbuf.at[pl.ds(0, sz)])

        nsb = sz // 128
        pend = {}
        dsems = (dsem0, dsem1)
        for j in range(nsb):
            par = j % 2
            if par in pend:
                pend[par].wait()
            for kk in range(8):
                didx[par, pl.ds(kk * 16, 16)] = dstbuf[
                    pl.ds(j * 128 + kk * 16, 16)]
            pend[par] = pltpu.async_copy(ones_v, degsp.at[didx.at[par]],
                                         dsems[par], add=True)
        tail = sz - nsb * 128
        if tail:
            par = nsb % 2
            if par in pend:
                pend[par].wait()
            dump16 = jnp.full((16,), N, jnp.int32)
            for kk in range(8):
                if kk * 16 < tail:
                    didx[par, pl.ds(kk * 16, 16)] = dstbuf[
                        pl.ds(nsb * 128 + kk * 16, 16)]
                else:
                    didx[par, pl.ds(kk * 16, 16)] = dump16
            pend[par] = pltpu.async_copy(ones_v, degsp.at[didx.at[par]],
                                         dsems[par], add=True)
        for par in pend:
            pend[par].wait()

        def _vreg(k, fills):
            f0, f1, f2, f3 = fills
            sv = srcbuf[pl.ds(k * 16, 16)]
            dv = dstbuf[pl.ds(k * 16, 16)]
            bid = ((dv >= BKT).astype(jnp.int32)
                   + (dv >= 2 * BKT).astype(jnp.int32)
                   + (dv >= 3 * BKT).astype(jnp.int32))
            dloc = dv - bid * BKT
            fl = [f0, f1, f2, f3]
            for b in range(NBKT):
                m = bid == b
                plsc.store_compressed(pend_s[b].at[pl.ds(fl[b], 16)], sv,
                                      mask=m)
                plsc.store_compressed(pend_d[b].at[pl.ds(fl[b], 16)], dloc,
                                      mask=m)
                fl[b] = fl[b] + jnp.sum(m.astype(jnp.int32))
            return tuple(fl)
        fills = lax.fori_loop(0, sz // 16, _vreg,
                              (rem[0], rem[1], rem[2], rem[3]))
        fills = list(fills)

        for b in range(NBKT):
            nfl = fills[b] // 128
            rbase = (w * NBKT + b) * NBATCH_CAP

            def _flush(k, nbb):
                bi = rbase + nbb
                pltpu.sync_copy(pend_s[b].at[pl.ds(k * 128, 128)],
                                sbdb_hbm.at[bi, 0])
                pltpu.sync_copy(pend_d[b].at[pl.ds(k * 128, 128)],
                                sbdb_hbm.at[bi, 1])
                return nbb + 1
            nb[b] = lax.fori_loop(0, nfl, _flush, nb[b])
            base = nfl * 128
            for kk in range(8):
                v = pend_s[b][pl.ds(base + kk * 16, 16)]
                pend_s[b][pl.ds(kk * 16, 16)] = v
                v2 = pend_d[b][pl.ds(base + kk * 16, 16)]
                pend_d[b][pl.ds(kk * 16, 16)] = v2
            fills[b] = fills[b] - nfl * 128
        return tuple(fills) + tuple(nb)

    zs = jnp.zeros((), jnp.int32)
    carry = (zs, zs, zs, zs, zs, zs, zs, zs)

    def _chunk(ci, carry):
        return process_chunk(w * EW + ci * CH, CH, carry)
    carry = lax.fori_loop(0, NFULL, _chunk, carry)
    carry = process_chunk(w * EW + NFULL * CH, TAIL, carry)
    rem = list(carry[0:4])
    nb = list(carry[4:8])

    nbv = jnp.zeros((16,), jnp.int32)
    dump16 = jnp.full((16,), DUMP, jnp.int32)
    z16i = jnp.zeros((16,), jnp.int32)
    for b in range(NBKT):
        for kk in range(8):
            idx = iota + kk * 16
            mpad = idx >= rem[b]
            v = pend_s[b][pl.ds(kk * 16, 16)]
            pend_s[b][pl.ds(kk * 16, 16)] = jnp.where(mpad, 0, v)
            v2 = pend_d[b][pl.ds(kk * 16, 16)]
            pend_d[b][pl.ds(kk * 16, 16)] = jnp.where(mpad, DUMP, v2)
            pend_s[b][pl.ds(128 + kk * 16, 16)] = z16i
            pend_d[b][pl.ds(128 + kk * 16, 16)] = dump16
        rbase = (w * NBKT + b) * NBATCH_CAP
        bi = rbase + nb[b]
        pltpu.sync_copy(pend_s[b].at[pl.ds(0, 128)], sbdb_hbm.at[bi, 0])
        pltpu.sync_copy(pend_d[b].at[pl.ds(0, 128)], sbdb_hbm.at[bi, 1])
        for t in range(8):
            pltpu.sync_copy(pend_s[b].at[pl.ds(128, 128)],
                            sbdb_hbm.at[bi + 1 + t, 0])
            pltpu.sync_copy(pend_d[b].at[pl.ds(128, 128)],
                            sbdb_hbm.at[bi + 1 + t, 1])
        nbv = nbv + jnp.where(iota == b, nb[b] + 1, 0)

    nbuf[...] = nbv
    pltpu.sync_copy(nbuf, nbt_hbm.at[w])

    plsc.subcore_barrier()
    pltpu.sync_copy(degsp.at[pl.ds(s * 6272, 6272)],
                    degp_hbm.at[c, pl.ds(s * 6272, 6272)])


def _make_sc_bin():
    mesh = plsc.VectorSubcoreMesh(core_axis_name="c", subcore_axis_name="s",
                                  num_cores=2, num_subcores=16)
    return functools.partial(
        pl.kernel,
        mesh=mesh,
        compiler_params=pltpu.CompilerParams(needs_layout_passes=False, use_tc_tiling_on_sc=False),
        out_type=[
            jax.ShapeDtypeStruct((NB3, 2, 128), jnp.int32),
            jax.ShapeDtypeStruct((NW, 16), jnp.int32),
            jax.ShapeDtypeStruct((2, NP), jnp.float32),
        ],
        scratch_types=[
            pltpu.VMEM((CH,), jnp.int32),
            pltpu.VMEM((CH,), jnp.int32),
            pltpu.VMEM((PCAP,), jnp.int32),
            pltpu.VMEM((PCAP,), jnp.int32),
            pltpu.VMEM((PCAP,), jnp.int32),
            pltpu.VMEM((PCAP,), jnp.int32),
            pltpu.VMEM((PCAP,), jnp.int32),
            pltpu.VMEM((PCAP,), jnp.int32),
            pltpu.VMEM((PCAP,), jnp.int32),
            pltpu.VMEM((PCAP,), jnp.int32),
            pltpu.VMEM((2, 128), jnp.int32),
            pltpu.VMEM((128,), jnp.float32),
            pltpu.VMEM((16,), jnp.int32),
            pltpu.VMEM((CH,), jnp.float32),
            pltpu.SemaphoreType.DMA,
            pltpu.SemaphoreType.DMA,
            pltpu.VMEM_SHARED((NP,), jnp.float32),
        ],
    )(_sc_bin_body)


def _sc_prop_body(zz_hbm, g_hbm, sbdb_hbm, nbt_hbm, out_hbm,
                  ib0, ib1, rows0, rows1, nbuf, sem0, sem1, sem2, sem3, acc):
    c = lax.axis_index("c")
    s = lax.axis_index("s")
    iota = lax.iota(jnp.int32, 16)

    for j in range(2):
        b = 2 * c + j
        pltpu.sync_copy(zz_hbm, rows0)
        for k in range(12):
            pltpu.sync_copy(rows0, acc.at[pl.ds(s * 1564 + k * 128, 128)])
        pltpu.sync_copy(rows0.at[pl.ds(0, 28)],
                        acc.at[pl.ds(s * 1564 + 12 * 128, 28)])
        plsc.subcore_barrier()

        for jj in range(2):
            wk = 2 * s + jj
            pltpu.sync_copy(nbt_hbm.at[wk], nbuf)
            nv = nbuf[pl.ds(0, 16)]
            nbatch = jnp.sum(jnp.where(iota == b, nv, 0))
            rbase = (wk * NBKT + b) * NBATCH_CAP

            def _pair(p, _):
                k0 = 2 * p
                k1 = jnp.minimum(k0 + 1, nbatch)
                pltpu.sync_copy(sbdb_hbm.at[k0 + rbase], ib0)
                cp0 = pltpu.async_copy(g_hbm.at[ib0.at[0]], rows0, sem0)
                pltpu.sync_copy(sbdb_hbm.at[k1 + rbase], ib1)
                cp1 = pltpu.async_copy(g_hbm.at[ib1.at[0]], rows1, sem1)
                cp0.wait()
                a0 = pltpu.async_copy(rows0, acc.at[ib0.at[1]], sem2, add=True)
                cp1.wait()
                a1 = pltpu.async_copy(rows1, acc.at[ib1.at[1]], sem3, add=True)
                a0.wait()
                a1.wait()
                return 0
            lax.fori_loop(0, (nbatch + 1) // 2, _pair, 0)

        plsc.subcore_barrier()
        pltpu.sync_copy(acc.at[pl.ds(s * 1560, 1560)],
                        out_hbm.at[pl.ds(b * BKT + s * 1560, 1560)])
        @pl.when(s == 15)
        def _():
            pltpu.sync_copy(acc.at[pl.ds(24960, 40)],
                            out_hbm.at[pl.ds(b * BKT + 24960, 40)])
        plsc.subcore_barrier()


def _make_sc_prop(dd):
    mesh = plsc.VectorSubcoreMesh(core_axis_name="c", subcore_axis_name="s",
                                  num_cores=2, num_subcores=16)
    return functools.partial(
        pl.kernel,
        mesh=mesh,
        compiler_params=pltpu.CompilerParams(needs_layout_passes=False, use_tc_tiling_on_sc=False),
        out_type=jax.ShapeDtypeStruct((NP, dd), jnp.float32),
        scratch_types=[
            pltpu.VMEM((2, 128), jnp.int32),
            pltpu.VMEM((2, 128), jnp.int32),
            pltpu.VMEM((128, dd), jnp.float32),
            pltpu.VMEM((128, dd), jnp.float32),
            pltpu.VMEM((16,), jnp.int32),
            pltpu.SemaphoreType.DMA,
            pltpu.SemaphoreType.DMA,
            pltpu.SemaphoreType.DMA,
            pltpu.SemaphoreType.DMA,
            pltpu.VMEM_SHARED((ACC_ROWS, dd), jnp.float32),
        ],
    )(_sc_prop_body)


def _prep_body(dp0, dp1, x, dinv_o, g0_o):
    deg = dp0[...] + dp1[...] + 1.0
    di = lax.rsqrt(deg)
    dinv_o[...] = di
    g0_o[...] = x[...] * di


def _prep(dp0, dp1, xp4):
    return pl.pallas_call(
        _prep_body,
        grid=(GRID,),
        in_specs=[
            pl.BlockSpec((RB, 1), lambda i: (i, 0)),
            pl.BlockSpec((RB, 1), lambda i: (i, 0)),
            pl.BlockSpec((RB, 16), lambda i: (i, 0)),
        ],
        out_specs=[
            pl.BlockSpec((RB, 1), lambda i: (i, 0)),
            pl.BlockSpec((RB, 16), lambda i: (i, 0)),
        ],
        out_shape=[
            jax.ShapeDtypeStruct((NP, 1), jnp.float32),
            jax.ShapeDtypeStruct((NP, 16), jnp.float32),
        ],
    )(dp0, dp1, xp4)


def _mid0_body(s_in, g_in, dinv, w0, wn, bc, g_next):
    i = pl.program_id(0)
    di = dinv[...]
    h = jnp.dot((s_in[...] + g_in[...]) * di, w0[...],
                preferred_element_type=jnp.float32) + bc[...]
    h = jnp.maximum(h, 0.0)
    rid = i * RB + lax.broadcasted_iota(jnp.int32, (RB, 1), 0)
    h = jnp.where(rid < N, h, 0.0)
    g_next[...] = jnp.dot(h, wn[...], preferred_element_type=jnp.float32) * di


def _mid0(s_arr, g_arr, dinv, w0p, wn, bc):
    return pl.pallas_call(
        _mid0_body,
        grid=(GRID,),
        in_specs=[
            pl.BlockSpec((RB, 16), lambda i: (i, 0)),
            pl.BlockSpec((RB, 16), lambda i: (i, 0)),
            pl.BlockSpec((RB, 1), lambda i: (i, 0)),
            pl.BlockSpec((16, H), lambda i: (0, 0)),
            pl.BlockSpec((H, H), lambda i: (0, 0)),
            pl.BlockSpec((1, H), lambda i: (0, 0)),
        ],
        out_specs=pl.BlockSpec((RB, H), lambda i: (i, 0)),
        out_shape=jax.ShapeDtypeStruct((NP, H), jnp.float32),
    )(s_arr, g_arr, dinv, w0p, wn, bc)


def _mid_body(s_in, g_in, dinv, wn, bc, g_next):
    i = pl.program_id(0)
    di = dinv[...]
    h = (s_in[...] + g_in[...]) * di + bc[...]
    h = jnp.maximum(h, 0.0)
    rid = i * RB + lax.broadcasted_iota(jnp.int32, (RB, 1), 0)
    h = jnp.where(rid < N, h, 0.0)
    g_next[...] = jnp.dot(h, wn[...], preferred_element_type=jnp.float32) * di


def _mid(s_arr, g_arr, dinv, wn, bc):
    return pl.pallas_call(
        _mid_body,
        grid=(GRID,),
        in_specs=[
            pl.BlockSpec((RB, H), lambda i: (i, 0)),
            pl.BlockSpec((RB, H), lambda i: (i, 0)),
            pl.BlockSpec((RB, 1), lambda i: (i, 0)),
            pl.BlockSpec((H, H), lambda i: (0, 0)),
            pl.BlockSpec((1, H), lambda i: (0, 0)),
        ],
        out_specs=pl.BlockSpec((RB, H), lambda i: (i, 0)),
        out_shape=jax.ShapeDtypeStruct((NP, H), jnp.float32),
    )(s_arr, g_arr, dinv, wn, bc)


def _pool_body(s_in, g_in, dinv, bc, bat, sums, maxs, cnts):
    i = pl.program_id(0)

    @pl.when(i == 0)
    def _():
        sums[...] = jnp.zeros((G, H), jnp.float32)
        maxs[...] = jnp.full((G, H), -3.0e38, jnp.float32)
        cnts[...] = jnp.zeros((G, 1), jnp.float32)

    h3 = (s_in[...] + g_in[...]) * dinv[...] + bc[...]
    rid = i * RB + lax.broadcasted_iota(jnp.int32, (RB, 1), 0)
    h3 = jnp.where(rid < N, h3, 0.0)
    b = bat[...]
    oh = (b == lax.broadcasted_iota(jnp.int32, (RB, G), 1)).astype(jnp.float32)
    sums[...] += lax.dot_general(oh, h3, (((0,), (0,)), ((), ())),
                                 preferred_element_type=jnp.float32)
    cnts[...] += lax.dot_general(oh, jnp.ones((RB, 1), jnp.float32),
                                 (((0,), (0,)), ((), ())),
                                 preferred_element_type=jnp.float32)
    glo = jnp.min(b)
    ghi = jnp.minimum(jnp.max(b), G - 1)

    def _seg(g, _):
        mcol = b == g
        vals = jnp.where(mcol, h3, -3.0e38)
        vmax = jnp.max(vals, axis=0, keepdims=True)
        cur = maxs[pl.ds(g, 1), :]
        maxs[pl.ds(g, 1), :] = jnp.maximum(cur, vmax)
        return 0
    lax.fori_loop(glo, ghi + 1, _seg, 0)


def _pool(s_arr, g_arr, dinv, bc, batp):
    return pl.pallas_call(
        _pool_body,
        grid=(GRID,),
        in_specs=[
            pl.BlockSpec((RB, H), lambda i: (i, 0)),
            pl.BlockSpec((RB, H), lambda i: (i, 0)),
            pl.BlockSpec((RB, 1), lambda i: (i, 0)),
            pl.BlockSpec((1, H), lambda i: (0, 0)),
            pl.BlockSpec((RB, 1), lambda i: (i, 0)),
        ],
        out_specs=[
            pl.BlockSpec((G, H), lambda i: (0, 0)),
            pl.BlockSpec((G, H), lambda i: (0, 0)),
            pl.BlockSpec((G, 1), lambda i: (0, 0)),
        ],
        out_shape=[
            jax.ShapeDtypeStruct((G, H), jnp.float32),
            jax.ShapeDtypeStruct((G, H), jnp.float32),
            jax.ShapeDtypeStruct((G, 1), jnp.float32),
        ],
    )(s_arr, g_arr, dinv, bc, batp)


def _head_body(sums, maxs, cnts, f1a, f1b, fb1, f2, fb2, out):
    cn = cnts[...]
    mean = sums[...] / jnp.maximum(cn, 1.0)
    mx = jnp.where(cn > 0.0, maxs[...], 0.0)
    a = jnp.dot(mean, f1a[...], preferred_element_type=jnp.float32)
    a += jnp.dot(mx, f1b[...], preferred_element_type=jnp.float32)
    a = jnp.maximum(a + fb1[...], 0.0)
    o = jnp.dot(a, f2[...], preferred_element_type=jnp.float32) + fb2[...]
    m = jnp.max(o, axis=1, keepdims=True)
    ex = jnp.exp(o - m)
    lse = jnp.log(jnp.sum(ex, axis=1, keepdims=True)) + m
    out[...] = o - lse


def _head(sums, maxs, cnts, f1a, f1b, fb1, f2, fb2):
    return pl.pallas_call(
        _head_body,
        out_shape=jax.ShapeDtypeStruct((G, C), jnp.float32),
    )(sums, maxs, cnts, f1a, f1b, fb1, f2, fb2)


def kernel(x, edge_index, batch, W0, b0, W1, b1, W2, b2,
           bn_gamma, bn_beta, bn_mean, bn_var, fc1_W, fc1_b, fc2_W, fc2_b):
    src = edge_index[0]
    dst = edge_index[1]

    gp = bn_gamma * lax.rsqrt(bn_var + 1e-5)
    Wt0 = W0 * gp[0][None, :]
    Wt1 = W1 * gp[1][None, :]
    Wt2 = W2 * gp[2][None, :]
    bt0 = ((b0 - bn_mean[0]) * gp[0] + bn_beta[0])[None, :]
    bt1 = ((b1 - bn_mean[1]) * gp[1] + bn_beta[1])[None, :]
    bt2 = ((b2 - bn_mean[2]) * gp[2] + bn_beta[2])[None, :]

    xp4 = jnp.pad(x, ((0, NP - N), (0, 13)))
    batp = jnp.pad(batch, (0, NP - N), constant_values=G).reshape(NP, 1)
    Wt0p = jnp.pad(Wt0, ((0, 13), (0, 0)))
    zz4 = jnp.zeros((128, 16), jnp.float32)
    zz64 = jnp.zeros((128, H), jnp.float32)

    sc_bin = _make_sc_bin()
    sc_prop4 = _make_sc_prop(16)
    sc_prop = _make_sc_prop(H)

    sbdb, nbt, degp = sc_bin(src, dst)
    dp0 = degp[0].reshape(NP, 1)
    dp1 = degp[1].reshape(NP, 1)
    dinv, g0x = _prep(dp0, dp1, xp4)

    s0 = sc_prop4(zz4, g0x, sbdb, nbt)
    g = _mid0(s0, g0x, dinv, Wt0p, Wt1, bt0)
    s1 = sc_prop(zz64, g, sbdb, nbt)
    g = _mid(s1, g, dinv, Wt2, bt1)
    s2 = sc_prop(zz64, g, sbdb, nbt)

    sums, maxs, cnts = _pool(s2, g, dinv, bt2, batp)
    out = _head(sums, maxs, cnts, fc1_W[:H], fc1_W[H:], fc1_b[None, :],
                fc2_W, fc2_b[None, :])
    return out


# speedup vs baseline: 1.2205x; 1.0238x over previous
import functools

import jax
import jax.numpy as jnp
from jax import lax
from jax.experimental import pallas as pl
from jax.experimental.pallas import tpu as pltpu
from jax.experimental.pallas import tpu_sc as plsc

N = 100000
E = 1600000
NP = 100352
H = 64
G = 64
C = 10

NBKT = 4
BKT = 25000
ACC_ROWS = 25024
DUMP = 25000
NW = 32
EW = E // NW
CH = 2048
NFULL = EW // CH
TAIL = EW - NFULL * CH
NBATCH_CAP = (EW + 127) // 128 + 9
NB3 = NW * NBKT * NBATCH_CAP
PCAP = 2304

RB = 2048
GRID = NP // RB


def _sc_bin_body(src_hbm, dst_hbm, sbdb_hbm, nbt_hbm, degp_hbm,
                 srcbuf, dstbuf, ps0, ps1, ps2, ps3, pd0, pd1, pd2, pd3,
                 didx, ones_v, nbuf, zbuf, dsem0, dsem1, degsp):
    pend_s = [ps0, ps1, ps2, ps3]
    pend_d = [pd0, pd1, pd2, pd3]
    c = lax.axis_index("c")
    s = lax.axis_index("s")
    w = 2 * s + c
    iota = lax.iota(jnp.int32, 16)
    z16f = jnp.zeros((16,), jnp.float32)

    def _zf(i, _):
        zbuf[pl.ds(i * 16, 16)] = z16f
        return 0
    lax.fori_loop(0, CH // 16, _zf, 0)
    for kk in range(3):
        pltpu.sync_copy(zbuf, degsp.at[pl.ds(s * 6272 + kk * CH, CH)])
    pltpu.sync_copy(zbuf.at[pl.ds(0, 128)],
                    degsp.at[pl.ds(s * 6272 + 3 * CH, 128)])
    for i in range(8):
        ones_v[pl.ds(i * 16, 16)] = jnp.ones((16,), jnp.float32)
    plsc.subcore_barrier()

    def process_chunk(estart, sz, carry):
        rem = list(carry[0:4])
        nb = list(carry[4:8])
        pltpu.sync_copy(src_hbm.at[pl.ds(estart, sz)], srcbuf.at[pl.ds(0, sz)])
        pltpu.sync_copy(dst_hbm.at[pl.ds(estart, sz)], dstbuf.at[pl.ds(0, sz)])

        nsb = sz // 128
        pend = {}
        dsems = (dsem0, dsem1)
        for j in range(nsb):
            par = j % 2
            if par in pend:
                pend[par].wait()
            for kk in range(8):
                didx[par, pl.ds(kk * 16, 16)] = dstbuf[
                    pl.ds(j * 128 + kk * 16, 16)]
            pend[par] = pltpu.async_copy(ones_v, degsp.at[didx.at[par]],
                                         dsems[par], add=True)
        tail = sz - nsb * 128
        if tail:
            par = nsb % 2
            if par in pend:
                pend[par].wait()
            dump16 = jnp.full((16,), N, jnp.int32)
            for kk in range(8):
                if kk * 16 < tail:
                    didx[par, pl.ds(kk * 16, 16)] = dstbuf[
                        pl.ds(nsb * 128 + kk * 16, 16)]
                else:
                    didx[par, pl.ds(kk * 16, 16)] = dump16
            pend[par] = pltpu.async_copy(ones_v, degsp.at[didx.at[par]],
                                         dsems[par], add=True)
        for par in pend:
            pend[par].wait()

        def _vreg(k, fills):
            f0, f1, f2, f3 = fills
            sv = srcbuf[pl.ds(k * 16, 16)]
            dv = dstbuf[pl.ds(k * 16, 16)]
            bid = ((dv >= BKT).astype(jnp.int32)
                   + (dv >= 2 * BKT).astype(jnp.int32)
                   + (dv >= 3 * BKT).astype(jnp.int32))
            dloc = dv - bid * BKT
            fl = [f0, f1, f2, f3]
            for b in range(NBKT):
                m = bid == b
                plsc.store_compressed(pend_s[b].at[pl.ds(fl[b], 16)], sv,
                                      mask=m)
                plsc.store_compressed(pend_d[b].at[pl.ds(fl[b], 16)], dloc,
                                      mask=m)
                fl[b] = fl[b] + jnp.sum(m.astype(jnp.int32))
            return tuple(fl)
        fills = lax.fori_loop(0, sz // 16, _vreg,
                              (rem[0], rem[1], rem[2], rem[3]))
        fills = list(fills)

        for b in range(NBKT):
            nfl = fills[b] // 128
            rbase = (w * NBKT + b) * NBATCH_CAP

            def _flush(k, nbb):
                bi = rbase + nbb
                pltpu.sync_copy(pend_s[b].at[pl.ds(k * 128, 128)],
                                sbdb_hbm.at[bi, 0])
                pltpu.sync_copy(pend_d[b].at[pl.ds(k * 128, 128)],
                                sbdb_hbm.at[bi, 1])
                return nbb + 1
            nb[b] = lax.fori_loop(0, nfl, _flush, nb[b])
            base = nfl * 128
            for kk in range(8):
                v = pend_s[b][pl.ds(base + kk * 16, 16)]
                pend_s[b][pl.ds(kk * 16, 16)] = v
                v2 = pend_d[b][pl.ds(base + kk * 16, 16)]
                pend_d[b][pl.ds(kk * 16, 16)] = v2
            fills[b] = fills[b] - nfl * 128
        return tuple(fills) + tuple(nb)

    zs = jnp.zeros((), jnp.int32)
    carry = (zs, zs, zs, zs, zs, zs, zs, zs)

    def _chunk(ci, carry):
        return process_chunk(w * EW + ci * CH, CH, carry)
    carry = lax.fori_loop(0, NFULL, _chunk, carry)
    carry = process_chunk(w * EW + NFULL * CH, TAIL, carry)
    rem = list(carry[0:4])
    nb = list(carry[4:8])

    nbv = jnp.zeros((16,), jnp.int32)
    dump16 = jnp.full((16,), DUMP, jnp.int32)
    z16i = jnp.zeros((16,), jnp.int32)
    for b in range(NBKT):
        for kk in range(8):
            idx = iota + kk * 16
            mpad = idx >= rem[b]
            v = pend_s[b][pl.ds(kk * 16, 16)]
            pend_s[b][pl.ds(kk * 16, 16)] = jnp.where(mpad, 0, v)
            v2 = pend_d[b][pl.ds(kk * 16, 16)]
            pend_d[b][pl.ds(kk * 16, 16)] = jnp.where(mpad, DUMP, v2)
            pend_s[b][pl.ds(128 + kk * 16, 16)] = z16i
            pend_d[b][pl.ds(128 + kk * 16, 16)] = dump16
        rbase = (w * NBKT + b) * NBATCH_CAP
        bi = rbase + nb[b]
        pltpu.sync_copy(pend_s[b].at[pl.ds(0, 128)], sbdb_hbm.at[bi, 0])
        pltpu.sync_copy(pend_d[b].at[pl.ds(0, 128)], sbdb_hbm.at[bi, 1])
        for t in range(8):
            pltpu.sync_copy(pend_s[b].at[pl.ds(128, 128)],
                            sbdb_hbm.at[bi + 1 + t, 0])
            pltpu.sync_copy(pend_d[b].at[pl.ds(128, 128)],
                            sbdb_hbm.at[bi + 1 + t, 1])
        nbv = nbv + jnp.where(iota == b, nb[b] + 1, 0)

    nbuf[...] = nbv
    pltpu.sync_copy(nbuf, nbt_hbm.at[w])

    plsc.subcore_barrier()
    pltpu.sync_copy(degsp.at[pl.ds(s * 6272, 6272)],
                    degp_hbm.at[c, pl.ds(s * 6272, 6272)])


def _make_sc_bin():
    mesh = plsc.VectorSubcoreMesh(core_axis_name="c", subcore_axis_name="s",
                                  num_cores=2, num_subcores=16)
    return functools.partial(
        pl.kernel,
        mesh=mesh,
        compiler_params=pltpu.CompilerParams(needs_layout_passes=False, use_tc_tiling_on_sc=False),
        out_type=[
            jax.ShapeDtypeStruct((NB3, 2, 128), jnp.int32),
            jax.ShapeDtypeStruct((NW, 16), jnp.int32),
            jax.ShapeDtypeStruct((2, NP), jnp.float32),
        ],
        scratch_types=[
            pltpu.VMEM((CH,), jnp.int32),
            pltpu.VMEM((CH,), jnp.int32),
            pltpu.VMEM((PCAP,), jnp.int32),
            pltpu.VMEM((PCAP,), jnp.int32),
            pltpu.VMEM((PCAP,), jnp.int32),
            pltpu.VMEM((PCAP,), jnp.int32),
            pltpu.VMEM((PCAP,), jnp.int32),
            pltpu.VMEM((PCAP,), jnp.int32),
            pltpu.VMEM((PCAP,), jnp.int32),
            pltpu.VMEM((PCAP,), jnp.int32),
            pltpu.VMEM((2, 128), jnp.int32),
            pltpu.VMEM((128,), jnp.float32),
            pltpu.VMEM((16,), jnp.int32),
            pltpu.VMEM((CH,), jnp.float32),
            pltpu.SemaphoreType.DMA,
            pltpu.SemaphoreType.DMA,
            pltpu.VMEM_SHARED((NP,), jnp.float32),
        ],
    )(_sc_bin_body)


def _sc_prop_body(zz_hbm, g_hbm, sbdb_hbm, nbt_hbm, out_hbm,
                  ib0, ib1, rows0, rows1, nbuf, sem0, sem1, sem2, sem3,
                  sem4, sem5, acc):
    c = lax.axis_index("c")
    s = lax.axis_index("s")
    iota = lax.iota(jnp.int32, 16)

    for j in range(2):
        b = 2 * c + j
        pltpu.sync_copy(zz_hbm, rows0)
        for k in range(12):
            pltpu.sync_copy(rows0, acc.at[pl.ds(s * 1564 + k * 128, 128)])
        pltpu.sync_copy(rows0.at[pl.ds(0, 28)],
                        acc.at[pl.ds(s * 1564 + 12 * 128, 28)])
        plsc.subcore_barrier()

        for jj in range(2):
            wk = 2 * s + jj
            pltpu.sync_copy(nbt_hbm.at[wk], nbuf)
            nv = nbuf[pl.ds(0, 16)]
            nbatch = jnp.sum(jnp.where(iota == b, nv, 0))
            rbase = (wk * NBKT + b) * NBATCH_CAP

            def _pair(p, _):
                k0 = 2 * p
                k1 = jnp.minimum(k0 + 1, nbatch)
                i0 = pltpu.async_copy(sbdb_hbm.at[k0 + rbase], ib0, sem4)
                i1 = pltpu.async_copy(sbdb_hbm.at[k1 + rbase], ib1, sem5)
                i0.wait()
                cp0 = pltpu.async_copy(g_hbm.at[ib0.at[0]], rows0, sem0)
                i1.wait()
                cp1 = pltpu.async_copy(g_hbm.at[ib1.at[0]], rows1, sem1)
                cp0.wait()
                a0 = pltpu.async_copy(rows0, acc.at[ib0.at[1]], sem2, add=True)
                cp1.wait()
                a1 = pltpu.async_copy(rows1, acc.at[ib1.at[1]], sem3, add=True)
                a0.wait()
                a1.wait()
                return 0
            lax.fori_loop(0, (nbatch + 1) // 2, _pair, 0)

        plsc.subcore_barrier()
        pltpu.sync_copy(acc.at[pl.ds(s * 1560, 1560)],
                        out_hbm.at[pl.ds(b * BKT + s * 1560, 1560)])
        @pl.when(s == 15)
        def _():
            pltpu.sync_copy(acc.at[pl.ds(24960, 40)],
                            out_hbm.at[pl.ds(b * BKT + 24960, 40)])
        plsc.subcore_barrier()


def _make_sc_prop(dd):
    mesh = plsc.VectorSubcoreMesh(core_axis_name="c", subcore_axis_name="s",
                                  num_cores=2, num_subcores=16)
    return functools.partial(
        pl.kernel,
        mesh=mesh,
        compiler_params=pltpu.CompilerParams(needs_layout_passes=False, use_tc_tiling_on_sc=False),
        out_type=jax.ShapeDtypeStruct((NP, dd), jnp.float32),
        scratch_types=[
            pltpu.VMEM((2, 128), jnp.int32),
            pltpu.VMEM((2, 128), jnp.int32),
            pltpu.VMEM((128, dd), jnp.float32),
            pltpu.VMEM((128, dd), jnp.float32),
            pltpu.VMEM((16,), jnp.int32),
            pltpu.SemaphoreType.DMA,
            pltpu.SemaphoreType.DMA,
            pltpu.SemaphoreType.DMA,
            pltpu.SemaphoreType.DMA,
            pltpu.SemaphoreType.DMA,
            pltpu.SemaphoreType.DMA,
            pltpu.VMEM_SHARED((ACC_ROWS, dd), jnp.float32),
        ],
    )(_sc_prop_body)


def _prep_body(dp0, dp1, x, dinv_o, g0_o):
    deg = dp0[...] + dp1[...] + 1.0
    di = lax.rsqrt(deg)
    dinv_o[...] = di
    g0_o[...] = x[...] * di


def _prep(dp0, dp1, xp4):
    return pl.pallas_call(
        _prep_body,
        grid=(GRID,),
        in_specs=[
            pl.BlockSpec((RB, 1), lambda i: (i, 0)),
            pl.BlockSpec((RB, 1), lambda i: (i, 0)),
            pl.BlockSpec((RB, 16), lambda i: (i, 0)),
        ],
        out_specs=[
            pl.BlockSpec((RB, 1), lambda i: (i, 0)),
            pl.BlockSpec((RB, 16), lambda i: (i, 0)),
        ],
        out_shape=[
            jax.ShapeDtypeStruct((NP, 1), jnp.float32),
            jax.ShapeDtypeStruct((NP, 16), jnp.float32),
        ],
    )(dp0, dp1, xp4)


def _mid0_body(s_in, g_in, dinv, w0, wn, bc, g_next):
    i = pl.program_id(0)
    di = dinv[...]
    h = jnp.dot((s_in[...] + g_in[...]) * di, w0[...],
                preferred_element_type=jnp.float32) + bc[...]
    h = jnp.maximum(h, 0.0)
    rid = i * RB + lax.broadcasted_iota(jnp.int32, (RB, 1), 0)
    h = jnp.where(rid < N, h, 0.0)
    g_next[...] = jnp.dot(h, wn[...], preferred_element_type=jnp.float32) * di


def _mid0(s_arr, g_arr, dinv, w0p, wn, bc):
    return pl.pallas_call(
        _mid0_body,
        grid=(GRID,),
        in_specs=[
            pl.BlockSpec((RB, 16), lambda i: (i, 0)),
            pl.BlockSpec((RB, 16), lambda i: (i, 0)),
            pl.BlockSpec((RB, 1), lambda i: (i, 0)),
            pl.BlockSpec((16, H), lambda i: (0, 0)),
            pl.BlockSpec((H, H), lambda i: (0, 0)),
            pl.BlockSpec((1, H), lambda i: (0, 0)),
        ],
        out_specs=pl.BlockSpec((RB, H), lambda i: (i, 0)),
        out_shape=jax.ShapeDtypeStruct((NP, H), jnp.float32),
    )(s_arr, g_arr, dinv, w0p, wn, bc)


def _mid_body(s_in, g_in, dinv, wn, bc, g_next):
    i = pl.program_id(0)
    di = dinv[...]
    h = (s_in[...] + g_in[...]) * di + bc[...]
    h = jnp.maximum(h, 0.0)
    rid = i * RB + lax.broadcasted_iota(jnp.int32, (RB, 1), 0)
    h = jnp.where(rid < N, h, 0.0)
    g_next[...] = jnp.dot(h, wn[...], preferred_element_type=jnp.float32) * di


def _mid(s_arr, g_arr, dinv, wn, bc):
    return pl.pallas_call(
        _mid_body,
        grid=(GRID,),
        in_specs=[
            pl.BlockSpec((RB, H), lambda i: (i, 0)),
            pl.BlockSpec((RB, H), lambda i: (i, 0)),
            pl.BlockSpec((RB, 1), lambda i: (i, 0)),
            pl.BlockSpec((H, H), lambda i: (0, 0)),
            pl.BlockSpec((1, H), lambda i: (0, 0)),
        ],
        out_specs=pl.BlockSpec((RB, H), lambda i: (i, 0)),
        out_shape=jax.ShapeDtypeStruct((NP, H), jnp.float32),
    )(s_arr, g_arr, dinv, wn, bc)


def _pool_body(s_in, g_in, dinv, bc, bat, f1a, f1b, fb1, f2, fb2,
               sums, maxs, cnts, out):
    i = pl.program_id(0)

    @pl.when(i == 0)
    def _():
        sums[...] = jnp.zeros((G, H), jnp.float32)
        maxs[...] = jnp.full((G, H), -3.0e38, jnp.float32)
        cnts[...] = jnp.zeros((G, 1), jnp.float32)

    h3 = (s_in[...] + g_in[...]) * dinv[...] + bc[...]
    rid = i * RB + lax.broadcasted_iota(jnp.int32, (RB, 1), 0)
    h3 = jnp.where(rid < N, h3, 0.0)
    b = bat[...]
    oh = (b == lax.broadcasted_iota(jnp.int32, (RB, G), 1)).astype(jnp.float32)
    sums[...] += lax.dot_general(oh, h3, (((0,), (0,)), ((), ())),
                                 preferred_element_type=jnp.float32)
    cnts[...] += lax.dot_general(oh, jnp.ones((RB, 1), jnp.float32),
                                 (((0,), (0,)), ((), ())),
                                 preferred_element_type=jnp.float32)
    glo = jnp.min(b)
    ghi = jnp.minimum(jnp.max(b), G - 1)

    def _seg(g, _):
        mcol = b == g
        vals = jnp.where(mcol, h3, -3.0e38)
        vmax = jnp.max(vals, axis=0, keepdims=True)
        cur = maxs[pl.ds(g, 1), :]
        maxs[pl.ds(g, 1), :] = jnp.maximum(cur, vmax)
        return 0
    lax.fori_loop(glo, ghi + 1, _seg, 0)

    @pl.when(i == GRID - 1)
    def _():
        cn = cnts[...]
        mean = sums[...] / jnp.maximum(cn, 1.0)
        mx = jnp.where(cn > 0.0, maxs[...], 0.0)
        a = jnp.dot(mean, f1a[...], preferred_element_type=jnp.float32)
        a += jnp.dot(mx, f1b[...], preferred_element_type=jnp.float32)
        a = jnp.maximum(a + fb1[...], 0.0)
        o = jnp.dot(a, f2[...], preferred_element_type=jnp.float32) + fb2[...]
        m = jnp.max(o, axis=1, keepdims=True)
        ex = jnp.exp(o - m)
        lse = jnp.log(jnp.sum(ex, axis=1, keepdims=True)) + m
        out[...] = o - lse


def _pool(s_arr, g_arr, dinv, bc, batp, f1a, f1b, fb1, f2, fb2):
    return pl.pallas_call(
        _pool_body,
        grid=(GRID,),
        in_specs=[
            pl.BlockSpec((RB, H), lambda i: (i, 0)),
            pl.BlockSpec((RB, H), lambda i: (i, 0)),
            pl.BlockSpec((RB, 1), lambda i: (i, 0)),
            pl.BlockSpec((1, H), lambda i: (0, 0)),
            pl.BlockSpec((RB, 1), lambda i: (i, 0)),
            pl.BlockSpec((H, H), lambda i: (0, 0)),
            pl.BlockSpec((H, H), lambda i: (0, 0)),
            pl.BlockSpec((1, H), lambda i: (0, 0)),
            pl.BlockSpec((H, C), lambda i: (0, 0)),
            pl.BlockSpec((1, C), lambda i: (0, 0)),
        ],
        out_specs=[
            pl.BlockSpec((G, H), lambda i: (0, 0)),
            pl.BlockSpec((G, H), lambda i: (0, 0)),
            pl.BlockSpec((G, 1), lambda i: (0, 0)),
            pl.BlockSpec((G, C), lambda i: (0, 0)),
        ],
        out_shape=[
            jax.ShapeDtypeStruct((G, H), jnp.float32),
            jax.ShapeDtypeStruct((G, H), jnp.float32),
            jax.ShapeDtypeStruct((G, 1), jnp.float32),
            jax.ShapeDtypeStruct((G, C), jnp.float32),
        ],
    )(s_arr, g_arr, dinv, bc, batp, f1a, f1b, fb1, f2, fb2)


def _head_body(sums, maxs, cnts, f1a, f1b, fb1, f2, fb2, out):
    cn = cnts[...]
    mean = sums[...] / jnp.maximum(cn, 1.0)
    mx = jnp.where(cn > 0.0, maxs[...], 0.0)
    a = jnp.dot(mean, f1a[...], preferred_element_type=jnp.float32)
    a += jnp.dot(mx, f1b[...], preferred_element_type=jnp.float32)
    a = jnp.maximum(a + fb1[...], 0.0)
    o = jnp.dot(a, f2[...], preferred_element_type=jnp.float32) + fb2[...]
    m = jnp.max(o, axis=1, keepdims=True)
    ex = jnp.exp(o - m)
    lse = jnp.log(jnp.sum(ex, axis=1, keepdims=True)) + m
    out[...] = o - lse


def _head(sums, maxs, cnts, f1a, f1b, fb1, f2, fb2):
    return pl.pallas_call(
        _head_body,
        out_shape=jax.ShapeDtypeStruct((G, C), jnp.float32),
    )(sums, maxs, cnts, f1a, f1b, fb1, f2, fb2)


def kernel(x, edge_index, batch, W0, b0, W1, b1, W2, b2,
           bn_gamma, bn_beta, bn_mean, bn_var, fc1_W, fc1_b, fc2_W, fc2_b):
    src = edge_index[0]
    dst = edge_index[1]

    gp = bn_gamma * lax.rsqrt(bn_var + 1e-5)
    Wt0 = W0 * gp[0][None, :]
    Wt1 = W1 * gp[1][None, :]
    Wt2 = W2 * gp[2][None, :]
    bt0 = ((b0 - bn_mean[0]) * gp[0] + bn_beta[0])[None, :]
    bt1 = ((b1 - bn_mean[1]) * gp[1] + bn_beta[1])[None, :]
    bt2 = ((b2 - bn_mean[2]) * gp[2] + bn_beta[2])[None, :]

    xp4 = jnp.pad(x, ((0, NP - N), (0, 13)))
    batp = jnp.pad(batch, (0, NP - N), constant_values=G).reshape(NP, 1)
    Wt0p = jnp.pad(Wt0, ((0, 13), (0, 0)))
    zz4 = jnp.zeros((128, 16), jnp.float32)
    zz64 = jnp.zeros((128, H), jnp.float32)

    sc_bin = _make_sc_bin()
    sc_prop4 = _make_sc_prop(16)
    sc_prop = _make_sc_prop(H)

    sbdb, nbt, degp = sc_bin(src, dst)
    dp0 = degp[0].reshape(NP, 1)
    dp1 = degp[1].reshape(NP, 1)
    dinv, g0x = _prep(dp0, dp1, xp4)

    s0 = sc_prop4(zz4, g0x, sbdb, nbt)
    g = _mid0(s0, g0x, dinv, Wt0p, Wt1, bt0)
    s1 = sc_prop(zz64, g, sbdb, nbt)
    g = _mid(s1, g, dinv, Wt2, bt1)
    s2 = sc_prop(zz64, g, sbdb, nbt)

    _, _, _, out = _pool(s2, g, dinv, bt2, batp, fc1_W[:H], fc1_W[H:],
                         fc1_b[None, :], fc2_W, fc2_b[None, :])
    return out
